# baseline (speedup 1.0000x reference)
import functools

import jax
import jax.numpy as jnp
from jax import lax
from jax.experimental import pallas as pl
from jax.experimental.pallas import tpu as pltpu
from jax.experimental.pallas import tpu_sc as plsc

DIM = 1024
HALF = 512
L0 = 64
L1 = 64
TOTAL = L0 * L1
NW = 32
LANES = 16
NCHUNK = HALF // LANES

SC_ROWS = 2048
R_W = SC_ROWS // NW
TC_BLK0 = SC_ROWS // L1
TC_NBLK = (TOTAL - SC_ROWS) // L1

_TWO_PI = 6.283185307179586
_PI = 3.141592653589793
_HALF_PI = 1.5707963267948966


def _sin_vec(x):
    y = x * (1.0 / _TWO_PI)
    k = jnp.where(y >= 0, y + 0.5, y - 0.5).astype(jnp.int32).astype(jnp.float32)
    r = x - k * _TWO_PI
    r = jnp.where(r > _HALF_PI, _PI - r, r)
    r = jnp.where(r < -_HALF_PI, -_PI - r, r)
    r2 = r * r
    p = jnp.float32(-2.5052108385441718e-08)
    p = p * r2 + 2.7557319223985893e-06
    p = p * r2 - 0.0001984126984126984
    p = p * r2 + 0.008333333333333333
    p = p * r2 - 0.16666666666666666
    p = p * r2 + 1.0
    return r * p


def _lane_broadcast(v, lane):
    idx = jnp.full((LANES, 1), lane, dtype=jnp.int32)
    dnums = lax.GatherDimensionNumbers(
        offset_dims=(), collapsed_slice_dims=(0,), start_index_map=(0,))
    return lax.gather(v, idx, dnums, (1,),
                      mode=lax.GatherScatterMode.PROMISE_IN_BOUNDS)


def _make_sc_kernel():
    mesh = plsc.VectorSubcoreMesh(core_axis_name="c", subcore_axis_name="s")

    @functools.partial(
        pl.kernel,
        mesh=mesh,
        out_type=jax.ShapeDtypeStruct((TOTAL, DIM), jnp.float32),
        scratch_types=[
            pltpu.VMEM((HALF,), jnp.float32),
            pltpu.VMEM((HALF,), jnp.float32),
            pltpu.VMEM((HALF,), jnp.float32),
            pltpu.VMEM((HALF,), jnp.float32),
            pltpu.VMEM((2 * LANES,), jnp.float32),
            pltpu.VMEM((HALF,), jnp.float32),
            pltpu.VMEM((HALF,), jnp.float32),
            pltpu.VMEM((HALF,), jnp.float32),
            pltpu.VMEM((HALF,), jnp.float32),
            pltpu.VMEM((HALF,), jnp.float32),
            pltpu.VMEM((R_W, HALF), jnp.float32),
            pltpu.VMEM((R_W, HALF), jnp.float32),
            pltpu.SemaphoreType.DMA,
            pltpu.SemaphoreType.DMA,
        ],
    )
    def sc_kernel(w0_h, b0_h, w1_h, b1_h, d0_h, m0_h, d1_h, m1_h, out_h,
                  w0_v, b0_v, w1_v, b1_v, scl_v, row_v,
                  sw_v, cw_v, s0_v, c0_v, rep, emb1,
                  sem_in, sem):
        wid = lax.axis_index("s") * 2 + lax.axis_index("c")
        row0 = wid * R_W

        cps = [
            pltpu.async_copy(w0_h, w0_v, sem_in),
            pltpu.async_copy(b0_h, b0_v, sem_in),
            pltpu.async_copy(w1_h, w1_v, sem_in),
            pltpu.async_copy(b1_h, b1_v, sem_in),
            pltpu.async_copy(d0_h, scl_v.at[pl.ds(0, 1)], sem_in),
            pltpu.async_copy(m0_h, scl_v.at[pl.ds(8, 1)], sem_in),
            pltpu.async_copy(d1_h, scl_v.at[pl.ds(16, 1)], sem_in),
            pltpu.async_copy(m1_h, scl_v.at[pl.ds(24, 1)], sem_in),
        ]
        for cp in cps:
            cp.wait()

        scl0 = scl_v[pl.ds(0, LANES)]
        scl1 = scl_v[pl.ds(LANES, LANES)]
        scale0 = _lane_broadcast(scl0, 8) / _lane_broadcast(scl0, 0)
        scale1 = _lane_broadcast(scl1, 8) / _lane_broadcast(scl1, 0)

        pos0 = ((R_W * wid) // L1).astype(jnp.float32)
        mstart = ((R_W * wid) % L1).astype(jnp.float32)

        def row_body(j, _):
            o = pl.multiple_of(j * LANES, LANES)
            w0c = w0_v[pl.ds(o, LANES)] * scale0
            row_v[pl.ds(o, LANES)] = _sin_vec(pos0 * w0c + b0_v[pl.ds(o, LANES)])
            a = w1_v[pl.ds(o, LANES)] * scale1
            th = b1_v[pl.ds(o, LANES)] + mstart * a
            sw_v[pl.ds(o, LANES)] = _sin_vec(a)
            cw_v[pl.ds(o, LANES)] = _sin_vec(a + _HALF_PI)
            s0_v[pl.ds(o, LANES)] = _sin_vec(th)
            c0_v[pl.ds(o, LANES)] = _sin_vec(th + _HALF_PI)
            return 0

        lax.fori_loop(0, NCHUNK, row_body, 0)

        vs = [row_v[pl.ds(j * LANES, LANES)] for j in range(NCHUNK)]

        def bc_body(r, _):
            for j in range(NCHUNK):
                rep[r, pl.ds(j * LANES, LANES)] = vs[j]
            return 0

        lax.fori_loop(0, R_W, bc_body, 0)
        dma_rep = pltpu.async_copy(
            rep, out_h.at[pl.ds(row0, R_W), pl.ds(0, HALF)], sem)

        GRP = 4

        def grp_body(jg, _):
            o = pl.multiple_of(jg * (GRP * LANES), GRP * LANES)
            ofs = [o + u * LANES for u in range(GRP)]
            sws = [sw_v[pl.ds(c, LANES)] for c in ofs]
            cws = [cw_v[pl.ds(c, LANES)] for c in ofs]
            ss = [s0_v[pl.ds(c, LANES)] for c in ofs]
            cs = [c0_v[pl.ds(c, LANES)] for c in ofs]
            for u in range(GRP):
                emb1[0, pl.ds(ofs[u], LANES)] = ss[u]

            def rec_body(r, carry):
                ss_c, cs_c = carry
                ns, nc = [], []
                for u in range(GRP):
                    s2 = ss_c[u] * cws[u] + cs_c[u] * sws[u]
                    c2 = cs_c[u] * cws[u] - ss_c[u] * sws[u]
                    emb1[r, pl.ds(ofs[u], LANES)] = s2
                    ns.append(s2)
                    nc.append(c2)
                return (tuple(ns), tuple(nc))

            lax.fori_loop(1, R_W, rec_body, (tuple(ss), tuple(cs)))
            return 0

        lax.fori_loop(0, NCHUNK // GRP, grp_body, 0)

        dma_emb = pltpu.async_copy(
            emb1, out_h.at[pl.ds(row0, R_W), pl.ds(HALF, HALF)], sem)

        dma_rep.wait()
        dma_emb.wait()

    return sc_kernel


_SC_KERNEL = _make_sc_kernel()


def _tc_body(d0_ref, m0_ref, d1_ref, m1_ref,
             a0_ref, b0_ref, a1_ref, b1_ref, part_ref, o_ref, emb1_scr):
    i = pl.program_id(0)
    sc0 = m0_ref[0] / d0_ref[0]
    sc1 = m1_ref[0] / d1_ref[0]

    @pl.when(i == 0)
    def _():
        c = lax.broadcasted_iota(jnp.int32, (L1, HALF), 0).astype(jnp.float32)
        emb1_scr[...] = jnp.sin((c * sc1) * a1_ref[:][None, :]
                                + b1_ref[:][None, :])

    c0 = (TC_BLK0 + i).astype(jnp.float32) * sc0
    row = jnp.sin(c0 * a0_ref[:] + b0_ref[:])
    o_ref[:, :HALF] = jnp.broadcast_to(row[None, :], (L1, HALF))
    o_ref[:, HALF:] = emb1_scr[...]


def _tc_fill(d0, m0, d1, m1, a0, b0, a1, b1, part):
    return pl.pallas_call(
        _tc_body,
        grid=(TC_NBLK,),
        in_specs=[
            pl.BlockSpec(memory_space=pltpu.SMEM),
            pl.BlockSpec(memory_space=pltpu.SMEM),
            pl.BlockSpec(memory_space=pltpu.SMEM),
            pl.BlockSpec(memory_space=pltpu.SMEM),
            pl.BlockSpec((HALF,), lambda i: (0,)),
            pl.BlockSpec((HALF,), lambda i: (0,)),
            pl.BlockSpec((HALF,), lambda i: (0,)),
            pl.BlockSpec((HALF,), lambda i: (0,)),
            pl.BlockSpec(memory_space=pl.ANY),
        ],
        out_specs=pl.BlockSpec((L1, DIM), lambda i: (TC_BLK0 + i, 0)),
        out_shape=jax.ShapeDtypeStruct((TOTAL, DIM), jnp.float32),
        scratch_shapes=[pltpu.VMEM((L1, HALF), jnp.float32)],
        input_output_aliases={8: 0},
    )(d0, m0, d1, m1, a0, b0, a1, b1, part)


def kernel(seq_len_or_axial_dims, W0, b0, W1, b1, div0, mult0, div1, mult1):
    w0f = jnp.reshape(W0, (HALF,))
    w1f = jnp.reshape(W1, (HALF,))
    d0 = jnp.reshape(div0, (1,))
    m0 = jnp.reshape(mult0, (1,))
    d1 = jnp.reshape(div1, (1,))
    m1 = jnp.reshape(mult1, (1,))
    part = _SC_KERNEL(w0f, b0, w1f, b1, d0, m0, d1, m1)
    return _tc_fill(d0, m0, d1, m1, w0f, b0, w1f, b1, part)
